# Initial kernel scaffold; baseline (speedup 1.0000x reference)
import jax
import jax.numpy as jnp
from jax.experimental import pallas as pl


def kernel(x, poly_ls, W0, W1):
    raise NotImplementedError("write your pallas kernel here")


# speedup vs baseline: 1.0281x; 1.0281x over previous
import functools

import jax
import jax.numpy as jnp
from jax.experimental import pallas as pl
from jax.experimental.pallas import tpu as pltpu


def _layer_body(a_ref, f_ref, w_ref, o_ref, b_ref, *, relu):
    @pl.when(pl.program_id(0) == 0)
    def _():
        f = f_ref[...]
        b_ref[0] = jnp.dot(f, w_ref[0], preferred_element_type=jnp.float32)
        b_ref[1] = jnp.dot(f, w_ref[1], preferred_element_type=jnp.float32)

    acc = jnp.dot(a_ref[0], b_ref[0], preferred_element_type=jnp.float32)
    acc = acc + jnp.dot(a_ref[1], b_ref[1], preferred_element_type=jnp.float32)
    if relu:
        acc = jnp.maximum(acc, 0.0)
    o_ref[...] = acc


def _poly_layer(poly_ls, feats, w, *, relu, block_m):
    p, n, _ = poly_ls.shape
    d_in = feats.shape[1]
    d_out = w.shape[2]
    grid = (n // block_m,)
    return pl.pallas_call(
        functools.partial(_layer_body, relu=relu),
        grid=grid,
        in_specs=[
            pl.BlockSpec((p, block_m, n), lambda i: (0, i, 0)),
            pl.BlockSpec((n, d_in), lambda i: (0, 0)),
            pl.BlockSpec((p, d_in, d_out), lambda i: (0, 0, 0)),
        ],
        out_specs=pl.BlockSpec((block_m, d_out), lambda i: (i, 0)),
        out_shape=jax.ShapeDtypeStruct((n, d_out), jnp.float32),
        scratch_shapes=[pltpu.VMEM((p, n, d_out), jnp.float32)],
    )(poly_ls, feats, w)


def kernel(x, poly_ls, W0, W1):
    n = x.shape[0]
    block_m = 200 if n % 200 == 0 else 8
    h = _poly_layer(poly_ls, x, W0, relu=True, block_m=block_m)
    return _poly_layer(poly_ls, h, W1, relu=False, block_m=block_m)
